# baseline (speedup 1.0000x reference)
import functools
import jax
import jax.numpy as jnp
from jax.experimental import pallas as pl
from jax.experimental.pallas import tpu as pltpu

_HIDDEN = 1024
_D_FF = 512
_E = 8
_T = 2048
_ROW_TILE = 128
_M_TILES = _T // _ROW_TILE
_ROUTED_SCALING = 2.5


def _sigmoid(v):
    return 1.0 / (1.0 + jnp.exp(-v))


def _routing_col(xm, router_w, e):
    logits = jax.lax.dot_general(
        xm, router_w, (((1,), (1,)), ((), ())),
        preferred_element_type=jnp.float32)
    s = _sigmoid(logits)
    cols = [s[:, i:i + 1] for i in range(_E)]
    g = [cols[2 * i] + cols[2 * i + 1] for i in range(4)]
    gmask = []
    for i in range(4):
        rank = jnp.zeros_like(g[i])
        for j in range(4):
            if j == i:
                continue
            gt = (g[j] > g[i]).astype(jnp.float32)
            eq_lower = ((g[j] == g[i]) & (j < i)).astype(jnp.float32)
            rank = rank + gt + eq_lower
        gmask.append(rank < 2.0)
    ms = [jnp.where(gmask[i // 2], cols[i], 0.0) for i in range(_E)]
    chosen = []
    for i in range(_E):
        rank = jnp.zeros_like(ms[i])
        for j in range(_E):
            if j == i:
                continue
            gt = (ms[j] > ms[i]).astype(jnp.float32)
            eq_lower = ((ms[j] == ms[i]) & (j < i)).astype(jnp.float32)
            rank = rank + gt + eq_lower
        chosen.append(rank < 2.0)
    wsum = jnp.zeros_like(cols[0])
    for i in range(_E):
        wsum = wsum + jnp.where(chosen[i], cols[i], 0.0)
    denom = wsum + 1e-20
    out = jnp.zeros_like(cols[0])
    for j in range(_E):
        rw_j = jnp.where(chosen[j], cols[j] / denom * _ROUTED_SCALING, 0.0)
        out = out + jnp.where(j == e, rw_j, 0.0)
    return out


def _mlp_block(xm, gw, uw, dw):
    gate = jax.lax.dot_general(xm, gw, (((1,), (1,)), ((), ())),
                               preferred_element_type=jnp.float32)
    up = jax.lax.dot_general(xm, uw, (((1,), (1,)), ((), ())),
                             preferred_element_type=jnp.float32)
    h = gate * _sigmoid(gate) * up
    return jax.lax.dot_general(h, dw, (((1,), (1,)), ((), ())),
                               preferred_element_type=jnp.float32)


def _moe_body(x_ref, rw_ref, gw_ref, uw_ref, dw_ref, sg_ref, su_ref,
              sd_ref, out_ref):
    e = pl.program_id(0)
    m = pl.program_id(1)
    rows = pl.ds(m * _ROW_TILE, _ROW_TILE)
    xm = x_ref[rows, :]
    w_col = _routing_col(xm, rw_ref[...], e)
    y = _mlp_block(xm, gw_ref[0], uw_ref[0], dw_ref[0]) * w_col

    @pl.when(e == 0)
    def _init():
        sh = _mlp_block(xm, sg_ref[...], su_ref[...], sd_ref[...])
        out_ref[rows, :] = y + sh

    @pl.when(e != 0)
    def _acc():
        out_ref[rows, :] += y


@jax.jit
def kernel(x, router_w, gate_w, up_w, down_w, s_gate, s_up, s_down):
    orig_shape = x.shape
    xf = x.reshape(_T, _HIDDEN)
    out = pl.pallas_call(
        _moe_body,
        grid=(_E, _M_TILES),
        in_specs=[
            pl.BlockSpec((_T, _HIDDEN), lambda e, m: (0, 0)),
            pl.BlockSpec((_E, _HIDDEN), lambda e, m: (0, 0)),
            pl.BlockSpec((1, _D_FF, _HIDDEN), lambda e, m: (e, 0, 0)),
            pl.BlockSpec((1, _D_FF, _HIDDEN), lambda e, m: (e, 0, 0)),
            pl.BlockSpec((1, _HIDDEN, _D_FF), lambda e, m: (e, 0, 0)),
            pl.BlockSpec((_D_FF, _HIDDEN), lambda e, m: (0, 0)),
            pl.BlockSpec((_D_FF, _HIDDEN), lambda e, m: (0, 0)),
            pl.BlockSpec((_HIDDEN, _D_FF), lambda e, m: (0, 0)),
        ],
        out_specs=pl.BlockSpec((_T, _HIDDEN), lambda e, m: (0, 0)),
        out_shape=jax.ShapeDtypeStruct((_T, _HIDDEN), jnp.float32),
        compiler_params=pltpu.CompilerParams(
            dimension_semantics=("arbitrary", "arbitrary")),
    )(xf, router_w, gate_w, up_w, down_w, s_gate, s_up, s_down)
    return out.reshape(orig_shape)


# speedup vs baseline: 1.4994x; 1.4994x over previous
import functools
import jax
import jax.numpy as jnp
from jax import lax
from jax.experimental import pallas as pl
from jax.experimental.pallas import tpu as pltpu
from jax.experimental.pallas import tpu_sc as plsc

_H = 1024
_F = 512
_E = 8
_T = 2048
_K = 2
_R = _T * _K
_TILE = 128
_M_TILES = _R // _TILE
_STEPS = _M_TILES + _E - 1
_SCALE = 2.5

_NC = 2
_NS = 16
_NW = _NC * _NS


def _sigmoid(v):
    return 1.0 / (1.0 + jnp.exp(-v))


def _router_body(x_ref, rw_ref, eid_ref, w_ref):
    logits = lax.dot_general(x_ref[...], rw_ref[...], (((1,), (1,)), ((), ())),
                             preferred_element_type=jnp.float32)
    s = _sigmoid(logits)
    cols = [s[:, i:i + 1] for i in range(_E)]
    g = [cols[2 * i] + cols[2 * i + 1] for i in range(4)]
    gmask = []
    for i in range(4):
        rank = jnp.zeros_like(g[i])
        for j in range(4):
            if j == i:
                continue
            rank = rank + (g[j] > g[i]).astype(jnp.float32)
            rank = rank + ((g[j] == g[i]) & (j < i)).astype(jnp.float32)
        gmask.append(rank < 2.0)
    ms = [jnp.where(gmask[i // 2], cols[i], 0.0) for i in range(_E)]
    ranks = []
    for i in range(_E):
        rank = jnp.zeros_like(ms[i])
        for j in range(_E):
            if j == i:
                continue
            rank = rank + (ms[j] > ms[i]).astype(jnp.float32)
            rank = rank + ((ms[j] == ms[i]) & (j < i)).astype(jnp.float32)
        ranks.append(rank)
    idx0 = jnp.zeros_like(cols[0])
    idx1 = jnp.zeros_like(cols[0])
    w0 = jnp.zeros_like(cols[0])
    w1 = jnp.zeros_like(cols[0])
    for i in range(_E):
        sel0 = ranks[i] == 0.0
        sel1 = ranks[i] == 1.0
        idx0 = jnp.where(sel0, float(i), idx0)
        idx1 = jnp.where(sel1, float(i), idx1)
        w0 = jnp.where(sel0, cols[i], w0)
        w1 = jnp.where(sel1, cols[i], w1)
    denom = w0 + w1 + 1e-20
    w0 = w0 / denom * _SCALE
    w1 = w1 / denom * _SCALE
    eid_ref[...] = jnp.concatenate([idx0, idx1], axis=1).astype(jnp.int32)
    w_ref[...] = jnp.concatenate([w0, w1], axis=1)


def _router(xf, router_w):
    return pl.pallas_call(
        _router_body,
        in_specs=[pl.BlockSpec((_T, _H), lambda: (0, 0)),
                  pl.BlockSpec((_E, _H), lambda: (0, 0))],
        out_specs=[pl.BlockSpec((_T, _K), lambda: (0, 0)),
                   pl.BlockSpec((_T, _K), lambda: (0, 0))],
        out_shape=[jax.ShapeDtypeStruct((_T, _K), jnp.int32),
                   jax.ShapeDtypeStruct((_T, _K), jnp.float32)],
    )(xf, router_w)


def _dispatch_body(eid_hbm, w_hbm, rt_hbm, rw_hbm, pos_hbm, cnt_hbm,
                   eid_v, w_v, dest_v, tok_v, cnt_v, acc_sm, prev_sm,
                   glob_sm, sem):
    c = lax.axis_index("c")
    s = lax.axis_index("s")
    lanes = lax.broadcasted_iota(jnp.int32, (16,), 0)
    base = s * 256

    @pl.when((c == 0) & (s == 0))
    def _zero():
        for e in range(_E):
            glob_sm[e] = 0

    plsc.subcore_barrier()

    @pl.when(c == 0)
    def _count():
        pltpu.sync_copy(eid_hbm.at[pl.ds(base, 256)], eid_v)
        pltpu.sync_copy(w_hbm.at[pl.ds(base, 256)], w_v)
        for e in range(_E):
            acc_sm[e] = 0

        def chunk_body(i, carry):
            ev = eid_v[pl.ds(i * 16, 16)]
            for l in range(16):
                e_l = ev[l]
                acc_sm[e_l] = acc_sm[e_l] + 1
            return carry

        lax.fori_loop(0, 16, chunk_body, 0)
        for e in range(_E):
            prev_sm[e] = plsc.fetch_and_add(glob_sm.at[e], acc_sm[e],
                                            subcore_id=0)

    plsc.subcore_barrier()

    @pl.when(c == 0)
    def _place():
        tot = []
        for e in range(_E):
            tot.append(plsc.fetch_and_add(glob_sm.at[e], 0, subcore_id=0))
        run = jnp.int32(0)
        for e in range(_E):
            acc_sm[e] = run + prev_sm[e]
            run = run + tot[e]

        def chunk_body(i, carry):
            ev = eid_v[pl.ds(i * 16, 16)]
            dest = jnp.zeros((16,), jnp.int32)
            for l in range(16):
                e_l = ev[l]
                d = acc_sm[e_l]
                acc_sm[e_l] = d + 1
                dest = dest + jnp.where(lanes == l, d, 0)
            row = i // 8
            col = (i % 8) * 16
            dest_v[row, pl.ds(col * 1, 16)] = dest
            tok_v[row, pl.ds(col * 1, 16)] = lax.shift_right_logical(
                base + i * 16 + lanes, 1)
            return carry

        lax.fori_loop(0, 16, chunk_body, 0)
        for j in range(2):
            pltpu.async_copy(tok_v.at[j], rt_hbm.at[dest_v.at[j]], sem).wait()
            pltpu.async_copy(w_v.at[pl.ds(j * 128, 128)],
                             rw_hbm.at[dest_v.at[j]], sem).wait()
        pltpu.sync_copy(dest_v, pos_hbm.at[pl.ds(s * 2, 2)])

        @pl.when(s == 0)
        def _counts_out():
            cntvec = jnp.zeros((16,), jnp.int32)
            for e in range(_E):
                cntvec = cntvec + jnp.where(lanes == e, tot[e], 0)
            cnt_v[...] = cntvec
            pltpu.sync_copy(cnt_v, cnt_hbm)


def _dispatch(eflat, wflat):
    mesh = plsc.VectorSubcoreMesh(core_axis_name="c", subcore_axis_name="s", num_cores=_NC, num_subcores=_NS)
    f = pl.kernel(
        _dispatch_body,
        out_type=[jax.ShapeDtypeStruct((_R,), jnp.int32),
                  jax.ShapeDtypeStruct((_R,), jnp.float32),
                  jax.ShapeDtypeStruct((2 * _NS, 128), jnp.int32),
                  jax.ShapeDtypeStruct((16,), jnp.int32)],
        mesh=mesh,
        scratch_types=[pltpu.VMEM((256,), jnp.int32),
                       pltpu.VMEM((256,), jnp.float32),
                       pltpu.VMEM((2, 128), jnp.int32),
                       pltpu.VMEM((2, 128), jnp.int32),
                       pltpu.VMEM((16,), jnp.int32),
                       pltpu.SMEM((_E,), jnp.int32),
                       pltpu.SMEM((_E,), jnp.int32),
                       pltpu.SMEM((_E,), jnp.int32),
                       pltpu.SemaphoreType.DMA],
    )
    return f(eflat, wflat)


def _gather_body(x_hbm, rt_hbm, xs_hbm, idx_v, rows_v, sem):
    c = lax.axis_index("c")
    s = lax.axis_index("s")
    wid = s * _NC + c
    rbase = wid * _TILE
    for k in range(2):
        pltpu.sync_copy(rt_hbm.at[pl.ds(rbase + k * 64, 64)], idx_v)
        pltpu.async_copy(x_hbm.at[idx_v], rows_v, sem).wait()
        pltpu.sync_copy(rows_v, xs_hbm.at[pl.ds(rbase + k * 64, 64)])


def _gather(xf, row_token):
    mesh = plsc.VectorSubcoreMesh(core_axis_name="c", subcore_axis_name="s", num_cores=_NC, num_subcores=_NS)
    f = pl.kernel(
        _gather_body,
        out_type=jax.ShapeDtypeStruct((_R, _H), jnp.float32),
        mesh=mesh,
        scratch_types=[pltpu.VMEM((64,), jnp.int32),
                       pltpu.VMEM((64, _H), jnp.float32),
                       pltpu.SemaphoreType.DMA],
    )
    return f(xf, row_token)


def _gmm_body(m_ref, g_ref, ss_ref, se_ref, zf_ref,
              xs_ref, gw_ref, uw_ref, dw_ref, rw_ref, out_ref):
    i = pl.program_id(0)
    xm = xs_ref[...]
    gate = lax.dot_general(xm, gw_ref[0], (((1,), (1,)), ((), ())),
                           preferred_element_type=jnp.float32)
    up = lax.dot_general(xm, uw_ref[0], (((1,), (1,)), ((), ())),
                         preferred_element_type=jnp.float32)
    h = gate * _sigmoid(gate) * up * rw_ref[...]
    y = lax.dot_general(h, dw_ref[0], (((1,), (1,)), ((), ())),
                        preferred_element_type=jnp.float32)
    rows = m_ref[i] * _TILE + lax.broadcasted_iota(jnp.int32, (_TILE, 1), 0)
    mask = (rows >= ss_ref[i]) & (rows < se_ref[i])
    y = jnp.where(mask, y, 0.0)

    @pl.when(zf_ref[i] == 1)
    def _init():
        out_ref[...] = y

    @pl.when(zf_ref[i] == 0)
    def _acc():
        out_ref[...] += y


def _gmm(xs, gate_w, up_w, down_w, row_w2d, m_ids, g_ids, seg_s, seg_e, zf):
    spec = pltpu.PrefetchScalarGridSpec(
        num_scalar_prefetch=5,
        grid=(_STEPS,),
        in_specs=[
            pl.BlockSpec((_TILE, _H), lambda i, m, g, ss, se, z: (m[i], 0)),
            pl.BlockSpec((1, _F, _H), lambda i, m, g, ss, se, z: (g[i], 0, 0)),
            pl.BlockSpec((1, _F, _H), lambda i, m, g, ss, se, z: (g[i], 0, 0)),
            pl.BlockSpec((1, _H, _F), lambda i, m, g, ss, se, z: (g[i], 0, 0)),
            pl.BlockSpec((_TILE, 1), lambda i, m, g, ss, se, z: (m[i], 0)),
        ],
        out_specs=pl.BlockSpec((_TILE, _H), lambda i, m, g, ss, se, z: (m[i], 0)),
    )
    return pl.pallas_call(
        _gmm_body,
        grid_spec=spec,
        out_shape=jax.ShapeDtypeStruct((_R, _H), jnp.float32),
        compiler_params=pltpu.CompilerParams(
            dimension_semantics=("arbitrary",)),
    )(m_ids, g_ids, seg_s, seg_e, zf, xs, gate_w, up_w, down_w, row_w2d)


def _shared_body(x_ref, sg_ref, su_ref, sd_ref, out_ref):
    xm = x_ref[...]
    gate = lax.dot_general(xm, sg_ref[...], (((1,), (1,)), ((), ())),
                           preferred_element_type=jnp.float32)
    up = lax.dot_general(xm, su_ref[...], (((1,), (1,)), ((), ())),
                         preferred_element_type=jnp.float32)
    h = gate * _sigmoid(gate) * up
    out_ref[...] = lax.dot_general(h, sd_ref[...], (((1,), (1,)), ((), ())),
                                   preferred_element_type=jnp.float32)


def _shared(xf, s_gate, s_up, s_down):
    return pl.pallas_call(
        _shared_body,
        grid=(_T // _TILE,),
        in_specs=[pl.BlockSpec((_TILE, _H), lambda m: (m, 0)),
                  pl.BlockSpec((_F, _H), lambda m: (0, 0)),
                  pl.BlockSpec((_F, _H), lambda m: (0, 0)),
                  pl.BlockSpec((_H, _F), lambda m: (0, 0))],
        out_specs=pl.BlockSpec((_TILE, _H), lambda m: (m, 0)),
        out_shape=jax.ShapeDtypeStruct((_T, _H), jnp.float32),
    )(xf, s_gate, s_up, s_down)


def _combine_body(ys_hbm, sh_hbm, pos_hbm, out_hbm, pidx_v, ybuf, obuf, sem):
    c = lax.axis_index("c")
    s = lax.axis_index("s")
    wid = s * _NC + c
    tbase = wid * 64
    for sub in range(2):
        t0 = tbase + sub * 32
        pltpu.sync_copy(pos_hbm.at[pl.ds(2 * t0, 64)], pidx_v)
        pltpu.async_copy(ys_hbm.at[pidx_v], ybuf, sem).wait()
        pltpu.sync_copy(sh_hbm.at[pl.ds(t0, 32)], obuf)

        def body(i, carry):
            for ch in range(_H // 16):
                sl = pl.ds(ch * 16, 16)
                obuf[i, sl] = obuf[i, sl] + ybuf[2 * i, sl] + ybuf[2 * i + 1, sl]
            return carry

        lax.fori_loop(0, 32, body, 0)
        pltpu.sync_copy(obuf, out_hbm.at[pl.ds(t0, 32)])


def _combine(ys, sh, pos_flat):
    mesh = plsc.VectorSubcoreMesh(core_axis_name="c", subcore_axis_name="s", num_cores=_NC, num_subcores=_NS)
    f = pl.kernel(
        _combine_body,
        out_type=jax.ShapeDtypeStruct((_T, _H), jnp.float32),
        mesh=mesh,
        scratch_types=[pltpu.VMEM((64,), jnp.int32),
                       pltpu.VMEM((64, _H), jnp.float32),
                       pltpu.VMEM((32, _H), jnp.float32),
                       pltpu.SemaphoreType.DMA],
    )
    return f(ys, sh, pos_flat)


def _metadata(counts):
    counts = counts[:_E].astype(jnp.int32)
    ends = jnp.cumsum(counts)
    starts = ends - counts
    nonempty = counts > 0
    first_tile = starts // _TILE
    t_g = jnp.where(nonempty, (ends - 1) // _TILE - first_tile + 1, 0)
    cum_t = jnp.cumsum(t_g)
    s_g = cum_t - t_g
    total = cum_t[_E - 1]
    i = jnp.arange(_STEPS, dtype=jnp.int32)
    gid = jnp.sum((cum_t[None, :] <= i[:, None]).astype(jnp.int32), axis=1)
    gid = jnp.minimum(gid, _E - 1)
    valid = i < total
    mi = first_tile[gid] + (i - s_g[gid])
    mi = jnp.where(valid, mi, _M_TILES - 1)
    seg_s = jnp.where(valid, starts[gid], 0)
    seg_e = jnp.where(valid, ends[gid], 0)
    prev = jnp.concatenate([jnp.array([-1], jnp.int32), mi[:-1]])
    zf = (valid & (mi != prev)).astype(jnp.int32)
    return mi, gid, seg_s, seg_e, zf


@jax.jit
def kernel(x, router_w, gate_w, up_w, down_w, s_gate, s_up, s_down):
    orig_shape = x.shape
    xf = x.reshape(_T, _H)
    eid2, w2 = _router(xf, router_w)
    eflat = eid2.reshape(_R)
    wflat = w2.reshape(_R)
    row_token, row_w, pos, counts = _dispatch(eflat, wflat)
    xs = _gather(xf, row_token)
    mi, gid, seg_s, seg_e, zf = _metadata(counts)
    ys = _gmm(xs, gate_w, up_w, down_w, row_w.reshape(_R, 1),
              mi, gid, seg_s, seg_e, zf)
    sh = _shared(xf, s_gate, s_up, s_down)
    out = _combine(ys, sh, pos.reshape(_R))
    return out.reshape(orig_shape)
